# baseline (device time: 213372 ns/iter reference)
import jax
import jax.numpy as jnp
from jax import lax
from jax.experimental import pallas as pl
from jax.experimental.pallas import tpu as pltpu

N_DEV = 8
H_PER = 8
DH = 128
SQ = 1024
SKV = 1024
D_MODEL = 1024
BLK = 64
SCALE = 0.08838834764831843


def kernel(x, Wq, K_ext, V_ext, Wo):
    i = lax.axis_index("i")

    xb = x[0].astype(jnp.bfloat16)
    wqb = Wq.astype(jnp.bfloat16)
    wob = Wo.astype(jnp.bfloat16)
    kb = lax.dynamic_slice_in_dim(K_ext[0], i * H_PER, H_PER, axis=1).astype(
        jnp.bfloat16
    )
    vb = lax.dynamic_slice_in_dim(V_ext[0], i * H_PER, H_PER, axis=1).astype(
        jnp.bfloat16
    )

    def body(x_ref, wq_ref, k_ref, v_ref, wo_ref, out_ref,
             ctx_ref, comm_ref, send_sems, recv_sems):
        my = lax.axis_index("i")
        left = lax.rem(my + N_DEV - 1, N_DEV)
        right = lax.rem(my + 1, N_DEV)

        barrier = pltpu.get_barrier_semaphore()
        for nbr in (left, right):
            pl.semaphore_signal(
                barrier, inc=1, device_id=(nbr,),
                device_id_type=pl.DeviceIdType.MESH,
            )
        pl.semaphore_wait(barrier, 2)

        q = jnp.dot(x_ref[...], wq_ref[...],
                    preferred_element_type=jnp.float32).astype(jnp.bfloat16)

        row_blk = lax.broadcasted_iota(jnp.int32, (SQ, SKV), 0) // BLK
        col_blk = lax.broadcasted_iota(jnp.int32, (SQ, SKV), 1) // BLK
        mask = col_blk <= row_blk

        for h in range(H_PER):
            qh = q[:, h * DH:(h + 1) * DH]
            kh = k_ref[:, h, :]
            s = lax.dot_general(qh, kh, (((1,), (1,)), ((), ())),
                                preferred_element_type=jnp.float32)
            s = jnp.where(mask, s * SCALE, -1e9)
            m = jnp.max(s, axis=-1, keepdims=True)
            e = jnp.exp(s - m)
            p = (e / jnp.sum(e, axis=-1, keepdims=True)).astype(jnp.bfloat16)
            ctx_ref[:, h * DH:(h + 1) * DH] = jnp.dot(
                p, v_ref[:, h, :], preferred_element_type=jnp.float32
            ).astype(jnp.bfloat16)

        partial = jnp.dot(ctx_ref[...], wo_ref[...],
                          preferred_element_type=jnp.float32)
        out_ref[0, :, :] = partial
        comm_ref[0, :, :] = partial.astype(jnp.bfloat16)

        for hop in range(N_DEV - 1):
            rdma = pltpu.make_async_remote_copy(
                src_ref=comm_ref.at[hop],
                dst_ref=comm_ref.at[hop + 1],
                send_sem=send_sems.at[hop],
                recv_sem=recv_sems.at[hop],
                device_id=(right,),
                device_id_type=pl.DeviceIdType.MESH,
            )
            rdma.start()
            rdma.wait()
            out_ref[0, :, :] = out_ref[0, :, :] + comm_ref[hop + 1].astype(
                jnp.float32
            )

    return pl.pallas_call(
        body,
        out_shape=jax.ShapeDtypeStruct((1, SQ, D_MODEL), jnp.float32),
        in_specs=[pl.BlockSpec(memory_space=pltpu.VMEM)] * 5,
        out_specs=pl.BlockSpec(memory_space=pltpu.VMEM),
        scratch_shapes=[
            pltpu.VMEM((SQ, H_PER * DH), jnp.bfloat16),
            pltpu.VMEM((N_DEV, SQ, D_MODEL), jnp.bfloat16),
            pltpu.SemaphoreType.DMA((N_DEV - 1,)),
            pltpu.SemaphoreType.DMA((N_DEV - 1,)),
        ],
        compiler_params=pltpu.CompilerParams(collective_id=0),
    )(xb, wqb, kb, vb, wob)


# device time: 91695 ns/iter; 2.3270x vs baseline; 2.3270x over previous
import jax
import jax.numpy as jnp
from jax import lax
from jax.experimental import pallas as pl
from jax.experimental.pallas import tpu as pltpu

N_DEV = 8
H_PER = 8
DH = 128
SQ = 1024
SKV = 1024
D_MODEL = 1024
BLK = 64
CHUNK = SQ // N_DEV
SCALE = 0.08838834764831843

_RS_HALF = (4, 2, 1)
_RS_OFF = (0, 4, 6)
_AG_SZ = (1, 2, 4)
_AG_OFF = (7, 8, 10)
_STAGE_CHUNKS = 14


def _gray(p):
    return jnp.bitwise_xor(p, jnp.bitwise_and(lax.shift_right_logical(p, 1), 1))


def kernel(x, Wq, K_ext, V_ext, Wo):
    i = lax.axis_index("i")

    xb = x[0].astype(jnp.bfloat16)
    wqb = Wq.astype(jnp.bfloat16)
    wob = Wo.astype(jnp.bfloat16)
    kb = lax.dynamic_slice_in_dim(K_ext[0], i * H_PER, H_PER, axis=1).astype(
        jnp.bfloat16
    )
    vb = lax.dynamic_slice_in_dim(V_ext[0], i * H_PER, H_PER, axis=1).astype(
        jnp.bfloat16
    )

    def body(x_ref, wq_ref, k_ref, v_ref, wo_ref, out_ref,
             ctx_ref, send_ref, recv_ref, send_sems, recv_sems):
        my = lax.axis_index("i")
        L = _gray(my)

        def partner(b):
            return _gray(jnp.bitwise_xor(L, 1 << b))

        barrier = pltpu.get_barrier_semaphore()
        for b in range(3):
            pl.semaphore_signal(
                barrier, inc=1, device_id=(partner(b),),
                device_id_type=pl.DeviceIdType.MESH,
            )
        pl.semaphore_wait(barrier, 3)

        q = jnp.dot(x_ref[...], wq_ref[...],
                    preferred_element_type=jnp.float32).astype(jnp.bfloat16)

        row_blk = lax.broadcasted_iota(jnp.int32, (SQ, SKV), 0) // BLK
        col_blk = lax.broadcasted_iota(jnp.int32, (SQ, SKV), 1) // BLK
        mask = col_blk <= row_blk

        for h in range(H_PER):
            qh = q[:, h * DH:(h + 1) * DH]
            kh = k_ref[:, h, :]
            s = lax.dot_general(qh, kh, (((1,), (1,)), ((), ())),
                                preferred_element_type=jnp.float32)
            s = jnp.where(mask, s * SCALE, -1e9)
            m = jnp.max(s, axis=-1, keepdims=True)
            e = jnp.exp(s - m)
            p = (e / jnp.sum(e, axis=-1, keepdims=True)).astype(jnp.bfloat16)
            ctx_ref[:, h * DH:(h + 1) * DH] = jnp.dot(
                p, v_ref[:, h, :], preferred_element_type=jnp.float32
            ).astype(jnp.bfloat16)

        out_ref[0, :, :] = jnp.dot(ctx_ref[...], wo_ref[...],
                                   preferred_element_type=jnp.float32)

        lo = lo0 = jnp.int32(0)
        lo = lo0
        for s, b in enumerate((2, 1, 0)):
            half = _RS_HALF[s]
            rows = half * CHUNK
            off = _RS_OFF[s] * CHUNK
            bit = jnp.bitwise_and(lax.shift_right_logical(L, b), 1)
            keep_lo = lo + bit * half
            send_lo = lo + (1 - bit) * half
            send_ref[pl.ds(off, rows), :] = out_ref[
                0, pl.ds(send_lo * CHUNK, rows), :
            ].astype(jnp.bfloat16)
            rdma = pltpu.make_async_remote_copy(
                src_ref=send_ref.at[pl.ds(off, rows), :],
                dst_ref=recv_ref.at[pl.ds(off, rows), :],
                send_sem=send_sems.at[s],
                recv_sem=recv_sems.at[s],
                device_id=(partner(b),),
                device_id_type=pl.DeviceIdType.MESH,
            )
            rdma.start()
            rdma.wait()
            out_ref[0, pl.ds(keep_lo * CHUNK, rows), :] = (
                out_ref[0, pl.ds(keep_lo * CHUNK, rows), :]
                + recv_ref[pl.ds(off, rows), :].astype(jnp.float32)
            )
            lo = keep_lo

        for s, b in enumerate((0, 1, 2)):
            sz = _AG_SZ[s]
            rows = sz * CHUNK
            off = _AG_OFF[s] * CHUNK
            recv_lo = jnp.bitwise_xor(lo, sz)
            send_ref[pl.ds(off, rows), :] = out_ref[
                0, pl.ds(lo * CHUNK, rows), :
            ].astype(jnp.bfloat16)
            rdma = pltpu.make_async_remote_copy(
                src_ref=send_ref.at[pl.ds(off, rows), :],
                dst_ref=recv_ref.at[pl.ds(off, rows), :],
                send_sem=send_sems.at[3 + s],
                recv_sem=recv_sems.at[3 + s],
                device_id=(partner(b),),
                device_id_type=pl.DeviceIdType.MESH,
            )
            rdma.start()
            rdma.wait()
            out_ref[0, pl.ds(recv_lo * CHUNK, rows), :] = recv_ref[
                pl.ds(off, rows), :
            ].astype(jnp.float32)
            lo = jnp.bitwise_and(lo, jnp.bitwise_not(jnp.int32(sz)))

    return pl.pallas_call(
        body,
        out_shape=jax.ShapeDtypeStruct((1, SQ, D_MODEL), jnp.float32),
        in_specs=[pl.BlockSpec(memory_space=pltpu.VMEM)] * 5,
        out_specs=pl.BlockSpec(memory_space=pltpu.VMEM),
        scratch_shapes=[
            pltpu.VMEM((SQ, H_PER * DH), jnp.bfloat16),
            pltpu.VMEM((_STAGE_CHUNKS * CHUNK, D_MODEL), jnp.bfloat16),
            pltpu.VMEM((_STAGE_CHUNKS * CHUNK, D_MODEL), jnp.bfloat16),
            pltpu.SemaphoreType.DMA((6,)),
            pltpu.SemaphoreType.DMA((6,)),
        ],
        compiler_params=pltpu.CompilerParams(collective_id=0),
    )(xb, wqb, kb, vb, wob)


# device time: 88046 ns/iter; 2.4234x vs baseline; 1.0414x over previous
import jax
import jax.numpy as jnp
from jax import lax
from jax.experimental import pallas as pl
from jax.experimental.pallas import tpu as pltpu

N_DEV = 8
H_PER = 8
DH = 128
SQ = 1024
SKV = 1024
D_MODEL = 1024
BLK = 64
CHUNK = SQ // N_DEV
SCALE = 0.08838834764831843

_RS_HALF = (4, 2, 1)
_RS_OFF = (0, 4, 6)
_AG_SZ = (1, 2, 4)
_AG_OFF = (7, 8, 10)
_STAGE_CHUNKS = 14


def _gray(p):
    return jnp.bitwise_xor(p, jnp.bitwise_and(lax.shift_right_logical(p, 1), 1))


def kernel(x, Wq, K_ext, V_ext, Wo):
    def body(x_ref, wq_ref, k_hbm, v_hbm, wo_ref, out_ref,
             ctx_ref, kv_ref, send_ref, recv_ref,
             dma_sems, send_sems, recv_sems):
        my = lax.axis_index("i")
        L = _gray(my)

        def partner(b):
            return _gray(jnp.bitwise_xor(L, 1 << b))

        k_dma = pltpu.make_async_copy(
            k_hbm.at[0, :, pl.ds(my * H_PER, H_PER), :], kv_ref.at[0],
            dma_sems.at[0],
        )
        v_dma = pltpu.make_async_copy(
            v_hbm.at[0, :, pl.ds(my * H_PER, H_PER), :], kv_ref.at[1],
            dma_sems.at[1],
        )
        k_dma.start()
        v_dma.start()

        barrier = pltpu.get_barrier_semaphore()
        for b in range(3):
            pl.semaphore_signal(
                barrier, inc=1, device_id=(partner(b),),
                device_id_type=pl.DeviceIdType.MESH,
            )
        pl.semaphore_wait(barrier, 3)

        q = (jnp.dot(x_ref[0].astype(jnp.bfloat16),
                     wq_ref[...].astype(jnp.bfloat16),
                     preferred_element_type=jnp.float32)
             * SCALE).astype(jnp.bfloat16)

        def blk_mask(qlo, nq, nk):
            rb = (lax.broadcasted_iota(jnp.int32, (nq, nk), 0) + qlo) // BLK
            cb = lax.broadcasted_iota(jnp.int32, (nq, nk), 1) // BLK
            return cb <= rb

        mask1 = blk_mask(0, SQ // 2, SKV // 2)
        mask2 = blk_mask(SQ // 2, SQ // 2, SKV)

        k_dma.wait()
        v_dma.wait()

        for h in range(H_PER):
            for (qlo, kn, mask) in ((0, SKV // 2, mask1), (SQ // 2, SKV, mask2)):
                qh = q[qlo:qlo + SQ // 2, h * DH:(h + 1) * DH]
                kh = kv_ref[0, :kn, h, :].astype(jnp.bfloat16)
                s = lax.dot_general(qh, kh, (((1,), (1,)), ((), ())),
                                    preferred_element_type=jnp.float32)
                e = jnp.where(mask, jnp.exp(s), 0.0)
                r = 1.0 / jnp.sum(e, axis=-1, keepdims=True)
                cc = jnp.dot(e.astype(jnp.bfloat16),
                             kv_ref[1, :kn, h, :].astype(jnp.bfloat16),
                             preferred_element_type=jnp.float32) * r
                ctx_ref[qlo:qlo + SQ // 2, h * DH:(h + 1) * DH] = cc.astype(
                    jnp.bfloat16)

        out_ref[0, :, :] = jnp.dot(ctx_ref[...],
                                   wo_ref[...].astype(jnp.bfloat16),
                                   preferred_element_type=jnp.float32)

        lo = jnp.int32(0)
        for s, b in enumerate((2, 1, 0)):
            half = _RS_HALF[s]
            rows = half * CHUNK
            off = _RS_OFF[s] * CHUNK
            bit = jnp.bitwise_and(lax.shift_right_logical(L, b), 1)
            keep_lo = lo + bit * half
            send_lo = lo + (1 - bit) * half
            send_ref[pl.ds(off, rows), :] = out_ref[
                0, pl.ds(send_lo * CHUNK, rows), :
            ].astype(jnp.bfloat16)
            rdma = pltpu.make_async_remote_copy(
                src_ref=send_ref.at[pl.ds(off, rows), :],
                dst_ref=recv_ref.at[pl.ds(off, rows), :],
                send_sem=send_sems.at[s],
                recv_sem=recv_sems.at[s],
                device_id=(partner(b),),
                device_id_type=pl.DeviceIdType.MESH,
            )
            rdma.start()
            rdma.wait()
            out_ref[0, pl.ds(keep_lo * CHUNK, rows), :] = (
                out_ref[0, pl.ds(keep_lo * CHUNK, rows), :]
                + recv_ref[pl.ds(off, rows), :].astype(jnp.float32)
            )
            lo = keep_lo

        for s, b in enumerate((0, 1, 2)):
            sz = _AG_SZ[s]
            rows = sz * CHUNK
            off = _AG_OFF[s] * CHUNK
            recv_lo = jnp.bitwise_xor(lo, sz)
            send_ref[pl.ds(off, rows), :] = out_ref[
                0, pl.ds(lo * CHUNK, rows), :
            ].astype(jnp.bfloat16)
            rdma = pltpu.make_async_remote_copy(
                src_ref=send_ref.at[pl.ds(off, rows), :],
                dst_ref=recv_ref.at[pl.ds(off, rows), :],
                send_sem=send_sems.at[3 + s],
                recv_sem=recv_sems.at[3 + s],
                device_id=(partner(b),),
                device_id_type=pl.DeviceIdType.MESH,
            )
            rdma.start()
            rdma.wait()
            out_ref[0, pl.ds(recv_lo * CHUNK, rows), :] = recv_ref[
                pl.ds(off, rows), :
            ].astype(jnp.float32)
            lo = jnp.bitwise_and(lo, jnp.bitwise_not(jnp.int32(sz)))

    return pl.pallas_call(
        body,
        out_shape=jax.ShapeDtypeStruct((1, SQ, D_MODEL), jnp.float32),
        in_specs=[
            pl.BlockSpec(memory_space=pltpu.VMEM),
            pl.BlockSpec(memory_space=pltpu.VMEM),
            pl.BlockSpec(memory_space=pltpu.MemorySpace.HBM),
            pl.BlockSpec(memory_space=pltpu.MemorySpace.HBM),
            pl.BlockSpec(memory_space=pltpu.VMEM),
        ],
        out_specs=pl.BlockSpec(memory_space=pltpu.VMEM),
        scratch_shapes=[
            pltpu.VMEM((SQ, H_PER * DH), jnp.bfloat16),
            pltpu.VMEM((2, SKV, H_PER, DH), jnp.float32),
            pltpu.VMEM((_STAGE_CHUNKS * CHUNK, D_MODEL), jnp.bfloat16),
            pltpu.VMEM((_STAGE_CHUNKS * CHUNK, D_MODEL), jnp.bfloat16),
            pltpu.SemaphoreType.DMA((2,)),
            pltpu.SemaphoreType.DMA((6,)),
            pltpu.SemaphoreType.DMA((6,)),
        ],
        compiler_params=pltpu.CompilerParams(collective_id=0),
    )(x, Wq, K_ext, V_ext, Wo)
